# baseline (speedup 1.0000x reference)
import dataclasses
import functools

import jax
import jax.numpy as jnp
from jax import lax
from jax.experimental import pallas as pl
from jax.experimental.pallas import tpu as pltpu
from jax.experimental.pallas import tpu_sc as plsc

N = 10000
E = 320000
G = 64
NTILES = 32
B = 400
NCH = 25
R = 2000
NBLK = N // R
RPT = N // 16
F32 = jnp.float32


def _proj_kernel(x_ref, wl_ref, bl_ref, wr_ref, br_ref, ol_ref, or_ref):
    xb = x_ref[...]
    ol_ref[...] = jnp.dot(xb, wl_ref[...], preferred_element_type=F32) + bl_ref[...]
    or_ref[...] = jnp.dot(xb, wr_ref[...], preferred_element_type=F32) + br_ref[...]


def _proj(xp, Wl, bl, Wr, br):
    K = xp.shape[1]
    return pl.pallas_call(
        _proj_kernel,
        grid=(NBLK,),
        in_specs=[
            pl.BlockSpec((R, K), lambda i: (i, 0)),
            pl.BlockSpec((K, 16), lambda i: (0, 0)),
            pl.BlockSpec((1, 16), lambda i: (0, 0)),
            pl.BlockSpec((K, 16), lambda i: (0, 0)),
            pl.BlockSpec((1, 16), lambda i: (0, 0)),
        ],
        out_specs=[
            pl.BlockSpec((R, 16), lambda i: (i, 0)),
            pl.BlockSpec((R, 16), lambda i: (i, 0)),
        ],
        out_shape=[jax.ShapeDtypeStruct((N, 16), F32)] * 2,
    )(xp, Wl, bl, Wr, br)


def _sc_agg(xl, xr, src3, dst3, attp):
    mesh = plsc.VectorSubcoreMesh(core_axis_name="c", subcore_axis_name="s")
    cp = pltpu.CompilerParams()
    if "needs_layout_passes" in pltpu.CompilerParams.__dataclass_fields__:
        cp = dataclasses.replace(cp, needs_layout_passes=False)
    if "use_tc_tiling_on_sc" in pltpu.CompilerParams.__dataclass_fields__:
        cp = dataclasses.replace(cp, use_tc_tiling_on_sc=False)

    @functools.partial(
        pl.kernel,
        out_type=jax.ShapeDtypeStruct((2 * N, 32), F32),
        mesh=mesh,
        compiler_params=cp,
        scratch_types=[
            pltpu.VMEM((NCH, B), jnp.int32),
            pltpu.VMEM((NCH, B), jnp.int32),
            pltpu.VMEM((16,), F32),
            pltpu.VMEM((B, 16), F32),
            pltpu.VMEM((B, 16), F32),
            pltpu.VMEM((B, 16), F32),
            pltpu.VMEM((B, 16), F32),
            pltpu.VMEM((B, 32), F32),
            pltpu.VMEM((B, 32), F32),
            pltpu.VMEM((RPT, 32), F32),
            pltpu.VMEM_SHARED((N, 32), F32),
            pltpu.VMEM_SHARED((N, 16), F32),
            pltpu.SemaphoreType.DMA,
            pltpu.SemaphoreType.DMA,
            pltpu.SemaphoreType.DMA,
            pltpu.SemaphoreType.DMA,
        ],
    )
    def k(xl_hbm, xr_hbm, src_hbm, dst_hbm, att_hbm, outd_hbm,
          srcv, dstv, attv, glv0, grv0, glv1, grv1, sbv0, sbv1, zv, shv,
          xlsh, sgl0, sgr0, sgl1, sgr1):
        c = lax.axis_index("c")
        s = lax.axis_index("s")
        wid = c * 16 + s

        @plsc.parallel_loop(0, RPT)
        def _zrow(i):
            zv[i, 0:16] = jnp.zeros((16,), F32)
            zv[i, 16:32] = jnp.zeros((16,), F32)

        pltpu.sync_copy(zv, shv.at[pl.ds(s * RPT, RPT)])
        pltpu.sync_copy(xl_hbm.at[pl.ds(s * RPT, RPT)], xlsh.at[pl.ds(s * RPT, RPT)])
        pltpu.sync_copy(src_hbm.at[wid], srcv)
        pltpu.sync_copy(dst_hbm.at[wid], dstv)
        pltpu.sync_copy(att_hbm, attv)
        plsc.subcore_barrier()
        attr = attv[...]

        def compute(glv, grv, sbv):
            @plsc.parallel_loop(0, B, unroll=8)
            def _edge(e):
                g = glv[e]
                r = grv[e]
                s2 = g + r
                p = jnp.maximum(s2, 0.2 * s2)
                t = p * attr
                a = jnp.sum(t)
                ev = jnp.exp(jnp.broadcast_to(a, (16,)))
                sbv[e, 0:16] = g * ev
                sbv[e, 16:32] = ev

        def issue(ci, glv, grv, sgl, sgr):
            pltpu.async_copy(xlsh.at[srcv.at[ci]], glv, sgl)
            pltpu.async_copy(xr_hbm.at[dstv.at[ci]], grv, sgr)

        def drain(ci, glv, grv, sgl, sgr):
            pltpu.make_async_copy(xlsh.at[srcv.at[ci]], glv, sgl).wait()
            pltpu.make_async_copy(xr_hbm.at[dstv.at[ci]], grv, sgr).wait()

        issue(0, glv0, grv0, sgl0, sgr0)

        @pl.loop(0, NCH - 1, step=2)
        def _pair(ci):
            drain(ci, glv0, grv0, sgl0, sgr0)
            issue(ci + 1, glv1, grv1, sgl1, sgr1)
            compute(glv0, grv0, sbv0)
            pltpu.sync_copy(sbv0, shv.at[dstv.at[ci]], add=True)
            drain(ci + 1, glv1, grv1, sgl1, sgr1)
            issue(ci + 2, glv0, grv0, sgl0, sgr0)
            compute(glv1, grv1, sbv1)
            pltpu.sync_copy(sbv1, shv.at[dstv.at[ci + 1]], add=True)

        drain(NCH - 1, glv0, grv0, sgl0, sgr0)
        compute(glv0, grv0, sbv0)
        pltpu.sync_copy(sbv0, shv.at[dstv.at[NCH - 1]], add=True)

        plsc.subcore_barrier()
        pltpu.sync_copy(shv.at[pl.ds(s * RPT, RPT)],
                        outd_hbm.at[pl.ds(c * N + s * RPT, RPT)])

    return k(xl, xr, src3, dst3, attp)


def _self_term(o1, o2, xl, xr, att_row):
    num = o1[:, 0:16] + o2[:, 0:16]
    den = o1[:, 16:17] + o2[:, 16:17]
    sl = xl + xr
    p = jnp.maximum(sl, 0.2 * sl)
    a = jnp.sum(p * att_row, axis=1, keepdims=True)
    e = jnp.exp(a)
    return num + e * xl, den + e


def _norm_proj_kernel(o1_ref, o2_ref, xl_ref, xr_ref, att_ref, b_ref,
                      wl_ref, bl_ref, wr_ref, br_ref, ol_ref, or_ref):
    num, den = _self_term(o1_ref[...], o2_ref[...], xl_ref[...], xr_ref[...],
                          att_ref[...])
    h = jnp.maximum(num / (den + 1e-16) + b_ref[...], 0.0)
    ol_ref[...] = jnp.dot(h, wl_ref[...], preferred_element_type=F32) + bl_ref[...]
    or_ref[...] = jnp.dot(h, wr_ref[...], preferred_element_type=F32) + br_ref[...]


def _norm_proj(outd, xl, xr, att, b1, Wl, bl, Wr, br):
    return pl.pallas_call(
        _norm_proj_kernel,
        grid=(NBLK,),
        in_specs=[
            pl.BlockSpec((R, 32), lambda i: (i, 0)),
            pl.BlockSpec((R, 32), lambda i: (i + NBLK, 0)),
            pl.BlockSpec((R, 16), lambda i: (i, 0)),
            pl.BlockSpec((R, 16), lambda i: (i, 0)),
            pl.BlockSpec((1, 16), lambda i: (0, 0)),
            pl.BlockSpec((1, 16), lambda i: (0, 0)),
            pl.BlockSpec((16, 16), lambda i: (0, 0)),
            pl.BlockSpec((1, 16), lambda i: (0, 0)),
            pl.BlockSpec((16, 16), lambda i: (0, 0)),
            pl.BlockSpec((1, 16), lambda i: (0, 0)),
        ],
        out_specs=[
            pl.BlockSpec((R, 16), lambda i: (i, 0)),
            pl.BlockSpec((R, 16), lambda i: (i, 0)),
        ],
        out_shape=[jax.ShapeDtypeStruct((N, 16), F32)] * 2,
    )(outd, outd, xl, xr, att, b1, Wl, bl, Wr, br)


def _norm_pool_kernel(o1_ref, o2_ref, xl_ref, xr_ref, att_ref, b_ref,
                      bt_ref, wlin_ref, blin_ref, s_ref, c_ref, out_ref):
    i = pl.program_id(0)
    num, den = _self_term(o1_ref[...], o2_ref[...], xl_ref[...], xr_ref[...],
                          att_ref[...])
    h = jnp.maximum(num / (den + 1e-16) + b_ref[...], 0.0)
    bt = jnp.reshape(bt_ref[...], (R, 1))
    labels = lax.broadcasted_iota(jnp.int32, (R, G), 1)
    oh = (bt == labels).astype(F32)
    contrib = lax.dot_general(oh, h, (((0,), (0,)), ((), ())),
                              preferred_element_type=F32)
    cnt = lax.dot_general(oh, jnp.ones((R, 1), F32), (((0,), (0,)), ((), ())),
                          preferred_element_type=F32)

    @pl.when(i == 0)
    def _():
        s_ref[...] = jnp.zeros_like(s_ref)
        c_ref[...] = jnp.zeros_like(c_ref)
        out_ref[...] = jnp.zeros_like(out_ref)

    s_ref[...] += contrib
    c_ref[...] += cnt

    @pl.when(i == NBLK - 1)
    def _():
        pooled = s_ref[...] / jnp.maximum(c_ref[...], 1.0)
        out_ref[...] = jnp.dot(pooled, wlin_ref[...],
                               preferred_element_type=F32) + blin_ref[...]


def _norm_pool(outd, xl, xr, att, b2, batchr, Wlinp, blin):
    _, _, out = pl.pallas_call(
        _norm_pool_kernel,
        grid=(NBLK,),
        in_specs=[
            pl.BlockSpec((R, 32), lambda i: (i, 0)),
            pl.BlockSpec((R, 32), lambda i: (i + NBLK, 0)),
            pl.BlockSpec((R, 16), lambda i: (i, 0)),
            pl.BlockSpec((R, 16), lambda i: (i, 0)),
            pl.BlockSpec((1, 16), lambda i: (0, 0)),
            pl.BlockSpec((1, 16), lambda i: (0, 0)),
            pl.BlockSpec((1, 1, R), lambda i: (i, 0, 0)),
            pl.BlockSpec((16, 1), lambda i: (0, 0)),
            pl.BlockSpec((1, 1), lambda i: (0, 0)),
        ],
        out_specs=[
            pl.BlockSpec((G, 16), lambda i: (0, 0)),
            pl.BlockSpec((G, 1), lambda i: (0, 0)),
            pl.BlockSpec((G, 1), lambda i: (0, 0)),
        ],
        out_shape=[
            jax.ShapeDtypeStruct((G, 16), F32),
            jax.ShapeDtypeStruct((G, 1), F32),
            jax.ShapeDtypeStruct((G, 1), F32),
        ],
    )(outd, outd, xl, xr, att, b2, batchr, Wlinp, blin)
    return out


def kernel(x, edge_index, batch, Wl1, bl1, Wr1, br1, att1, b1,
           Wl2, bl2, Wr2, br2, att2, b2, Wlin, blin):
    src3 = jnp.reshape(edge_index[0], (NTILES, NCH, B))
    dst3 = jnp.reshape(edge_index[1], (NTILES, NCH, B))
    batchr = jnp.reshape(batch, (NBLK, 1, R))

    bl1r = jnp.reshape(bl1, (1, 16))
    br1r = jnp.reshape(br1, (1, 16))
    b1r = jnp.reshape(b1, (1, 16))
    att1r = jnp.reshape(att1, (1, 16))
    b2r = jnp.reshape(jnp.pad(b2, (0, 8)), (1, 16))
    att2p = jnp.pad(att2, (0, 8))
    att2r = jnp.reshape(att2p, (1, 16))
    Wl2p = jnp.pad(Wl2, ((0, 0), (0, 8)))
    Wr2p = jnp.pad(Wr2, ((0, 0), (0, 8)))
    bl2r = jnp.reshape(jnp.pad(bl2, (0, 8)), (1, 16))
    br2r = jnp.reshape(jnp.pad(br2, (0, 8)), (1, 16))
    Wlinp = jnp.pad(Wlin, ((0, 8), (0, 0)))
    blinr = jnp.reshape(blin, (1, 1))

    xl1, xr1 = _proj(x, Wl1, bl1r, Wr1, br1r)
    outd1 = _sc_agg(xl1, xr1, src3, dst3, att1)
    xl2, xr2 = _norm_proj(outd1, xl1, xr1, att1r, b1r, Wl2p, bl2r, Wr2p, br2r)
    outd2 = _sc_agg(xl2, xr2, src3, dst3, att2p)
    return _norm_pool(outd2, xl2, xr2, att2r, b2r, batchr, Wlinp, blinr)


# speedup vs baseline: 1.1415x; 1.1415x over previous
import dataclasses
import functools

import jax
import jax.numpy as jnp
from jax import lax
from jax.experimental import pallas as pl
from jax.experimental.pallas import tpu as pltpu
from jax.experimental.pallas import tpu_sc as plsc

N = 10000
E = 320000
G = 64
NTILES = 32
B = 400
NCH = 25
R = 2000
NBLK = N // R
RPT = N // 16
F32 = jnp.float32


def _proj_kernel(x_ref, wl_ref, bl_ref, wr_ref, br_ref, ol_ref, or_ref):
    xb = x_ref[...]
    ol_ref[...] = jnp.dot(xb, wl_ref[...], preferred_element_type=F32) + bl_ref[...]
    or_ref[...] = jnp.dot(xb, wr_ref[...], preferred_element_type=F32) + br_ref[...]


def _proj(xp, Wl, bl, Wr, br):
    K = xp.shape[1]
    return pl.pallas_call(
        _proj_kernel,
        grid=(NBLK,),
        in_specs=[
            pl.BlockSpec((R, K), lambda i: (i, 0)),
            pl.BlockSpec((K, 16), lambda i: (0, 0)),
            pl.BlockSpec((1, 16), lambda i: (0, 0)),
            pl.BlockSpec((K, 16), lambda i: (0, 0)),
            pl.BlockSpec((1, 16), lambda i: (0, 0)),
        ],
        out_specs=[
            pl.BlockSpec((R, 16), lambda i: (i, 0)),
            pl.BlockSpec((R, 16), lambda i: (i, 0)),
        ],
        out_shape=[jax.ShapeDtypeStruct((N, 16), F32)] * 2,
    )(xp, Wl, bl, Wr, br)


def _sc_agg(xl, xr, src3, dst3, attp):
    mesh = plsc.VectorSubcoreMesh(core_axis_name="c", subcore_axis_name="s")
    cp = pltpu.CompilerParams()
    if "needs_layout_passes" in pltpu.CompilerParams.__dataclass_fields__:
        cp = dataclasses.replace(cp, needs_layout_passes=False)
    if "use_tc_tiling_on_sc" in pltpu.CompilerParams.__dataclass_fields__:
        cp = dataclasses.replace(cp, use_tc_tiling_on_sc=False)

    @functools.partial(
        pl.kernel,
        out_type=jax.ShapeDtypeStruct((2 * N, 32), F32),
        mesh=mesh,
        compiler_params=cp,
        scratch_types=[
            pltpu.VMEM((NCH, B), jnp.int32),
            pltpu.VMEM((NCH, B), jnp.int32),
            pltpu.VMEM((16,), F32),
            pltpu.VMEM((B, 16), F32),
            pltpu.VMEM((B, 16), F32),
            pltpu.VMEM((B, 16), F32),
            pltpu.VMEM((B, 16), F32),
            pltpu.VMEM((B, 32), F32),
            pltpu.VMEM((B, 32), F32),
            pltpu.VMEM((RPT, 32), F32),
            pltpu.VMEM_SHARED((N, 32), F32),
            pltpu.SemaphoreType.DMA,
            pltpu.SemaphoreType.DMA,
            pltpu.SemaphoreType.DMA,
            pltpu.SemaphoreType.DMA,
        ],
    )
    def k(xl_hbm, xr_hbm, src_hbm, dst_hbm, att_hbm, outd_hbm,
          srcv, dstv, attv, glv0, grv0, glv1, grv1, sbv0, sbv1, zv, shv,
          sgl0, sgr0, sgl1, sgr1):
        c = lax.axis_index("c")
        s = lax.axis_index("s")
        wid = c * 16 + s

        @plsc.parallel_loop(0, RPT)
        def _zrow(i):
            zv[i, 0:16] = jnp.zeros((16,), F32)
            zv[i, 16:32] = jnp.zeros((16,), F32)

        pltpu.sync_copy(zv, shv.at[pl.ds(s * RPT, RPT)])
        pltpu.sync_copy(src_hbm.at[wid], srcv)
        pltpu.sync_copy(dst_hbm.at[wid], dstv)
        pltpu.sync_copy(att_hbm, attv)
        plsc.subcore_barrier()
        attr = attv[...]

        def compute(glv, grv, sbv):
            @plsc.parallel_loop(0, B, unroll=8)
            def _edge(e):
                g = glv[e]
                r = grv[e]
                s2 = g + r
                p = jnp.maximum(s2, 0.2 * s2)
                t = p * attr
                a = jnp.sum(t)
                ev = jnp.exp(jnp.broadcast_to(a, (16,)))
                sbv[e, 0:16] = g * ev
                sbv[e, 16:32] = ev

        def issue(ci, glv, grv, sgl, sgr):
            pltpu.async_copy(xl_hbm.at[srcv.at[ci]], glv, sgl)
            pltpu.async_copy(xr_hbm.at[dstv.at[ci]], grv, sgr)

        def drain(ci, glv, grv, sgl, sgr):
            pltpu.make_async_copy(xl_hbm.at[srcv.at[ci]], glv, sgl).wait()
            pltpu.make_async_copy(xr_hbm.at[dstv.at[ci]], grv, sgr).wait()

        issue(0, glv0, grv0, sgl0, sgr0)

        @pl.loop(0, NCH - 1, step=2)
        def _pair(ci):
            drain(ci, glv0, grv0, sgl0, sgr0)
            issue(ci + 1, glv1, grv1, sgl1, sgr1)
            compute(glv0, grv0, sbv0)
            pltpu.sync_copy(sbv0, shv.at[dstv.at[ci]], add=True)
            drain(ci + 1, glv1, grv1, sgl1, sgr1)
            issue(ci + 2, glv0, grv0, sgl0, sgr0)
            compute(glv1, grv1, sbv1)
            pltpu.sync_copy(sbv1, shv.at[dstv.at[ci + 1]], add=True)

        drain(NCH - 1, glv0, grv0, sgl0, sgr0)
        compute(glv0, grv0, sbv0)
        pltpu.sync_copy(sbv0, shv.at[dstv.at[NCH - 1]], add=True)

        plsc.subcore_barrier()
        pltpu.sync_copy(shv.at[pl.ds(s * RPT, RPT)],
                        outd_hbm.at[pl.ds(c * N + s * RPT, RPT)])

    return k(xl, xr, src3, dst3, attp)


def _self_term(o1, o2, xl, xr, att_row):
    num = o1[:, 0:16] + o2[:, 0:16]
    den = o1[:, 16:17] + o2[:, 16:17]
    sl = xl + xr
    p = jnp.maximum(sl, 0.2 * sl)
    a = jnp.sum(p * att_row, axis=1, keepdims=True)
    e = jnp.exp(a)
    return num + e * xl, den + e


def _norm_proj_kernel(o1_ref, o2_ref, xl_ref, xr_ref, att_ref, b_ref,
                      wl_ref, bl_ref, wr_ref, br_ref, ol_ref, or_ref):
    num, den = _self_term(o1_ref[...], o2_ref[...], xl_ref[...], xr_ref[...],
                          att_ref[...])
    h = jnp.maximum(num / (den + 1e-16) + b_ref[...], 0.0)
    ol_ref[...] = jnp.dot(h, wl_ref[...], preferred_element_type=F32) + bl_ref[...]
    or_ref[...] = jnp.dot(h, wr_ref[...], preferred_element_type=F32) + br_ref[...]


def _norm_proj(outd, xl, xr, att, b1, Wl, bl, Wr, br):
    return pl.pallas_call(
        _norm_proj_kernel,
        grid=(NBLK,),
        in_specs=[
            pl.BlockSpec((R, 32), lambda i: (i, 0)),
            pl.BlockSpec((R, 32), lambda i: (i + NBLK, 0)),
            pl.BlockSpec((R, 16), lambda i: (i, 0)),
            pl.BlockSpec((R, 16), lambda i: (i, 0)),
            pl.BlockSpec((1, 16), lambda i: (0, 0)),
            pl.BlockSpec((1, 16), lambda i: (0, 0)),
            pl.BlockSpec((16, 16), lambda i: (0, 0)),
            pl.BlockSpec((1, 16), lambda i: (0, 0)),
            pl.BlockSpec((16, 16), lambda i: (0, 0)),
            pl.BlockSpec((1, 16), lambda i: (0, 0)),
        ],
        out_specs=[
            pl.BlockSpec((R, 16), lambda i: (i, 0)),
            pl.BlockSpec((R, 16), lambda i: (i, 0)),
        ],
        out_shape=[jax.ShapeDtypeStruct((N, 16), F32)] * 2,
    )(outd, outd, xl, xr, att, b1, Wl, bl, Wr, br)


def _norm_pool_kernel(o1_ref, o2_ref, xl_ref, xr_ref, att_ref, b_ref,
                      bt_ref, wlin_ref, blin_ref, s_ref, c_ref, out_ref):
    i = pl.program_id(0)
    num, den = _self_term(o1_ref[...], o2_ref[...], xl_ref[...], xr_ref[...],
                          att_ref[...])
    h = jnp.maximum(num / (den + 1e-16) + b_ref[...], 0.0)
    bt = jnp.reshape(bt_ref[...], (R, 1))
    labels = lax.broadcasted_iota(jnp.int32, (R, G), 1)
    oh = (bt == labels).astype(F32)
    contrib = lax.dot_general(oh, h, (((0,), (0,)), ((), ())),
                              preferred_element_type=F32)
    cnt = lax.dot_general(oh, jnp.ones((R, 1), F32), (((0,), (0,)), ((), ())),
                          preferred_element_type=F32)

    @pl.when(i == 0)
    def _():
        s_ref[...] = jnp.zeros_like(s_ref)
        c_ref[...] = jnp.zeros_like(c_ref)
        out_ref[...] = jnp.zeros_like(out_ref)

    s_ref[...] += contrib
    c_ref[...] += cnt

    @pl.when(i == NBLK - 1)
    def _():
        pooled = s_ref[...] / jnp.maximum(c_ref[...], 1.0)
        out_ref[...] = jnp.dot(pooled, wlin_ref[...],
                               preferred_element_type=F32) + blin_ref[...]


def _norm_pool(outd, xl, xr, att, b2, batchr, Wlinp, blin):
    _, _, out = pl.pallas_call(
        _norm_pool_kernel,
        grid=(NBLK,),
        in_specs=[
            pl.BlockSpec((R, 32), lambda i: (i, 0)),
            pl.BlockSpec((R, 32), lambda i: (i + NBLK, 0)),
            pl.BlockSpec((R, 16), lambda i: (i, 0)),
            pl.BlockSpec((R, 16), lambda i: (i, 0)),
            pl.BlockSpec((1, 16), lambda i: (0, 0)),
            pl.BlockSpec((1, 16), lambda i: (0, 0)),
            pl.BlockSpec((1, 1, R), lambda i: (i, 0, 0)),
            pl.BlockSpec((16, 1), lambda i: (0, 0)),
            pl.BlockSpec((1, 1), lambda i: (0, 0)),
        ],
        out_specs=[
            pl.BlockSpec((G, 16), lambda i: (0, 0)),
            pl.BlockSpec((G, 1), lambda i: (0, 0)),
            pl.BlockSpec((G, 1), lambda i: (0, 0)),
        ],
        out_shape=[
            jax.ShapeDtypeStruct((G, 16), F32),
            jax.ShapeDtypeStruct((G, 1), F32),
            jax.ShapeDtypeStruct((G, 1), F32),
        ],
    )(outd, outd, xl, xr, att, b2, batchr, Wlinp, blin)
    return out


def kernel(x, edge_index, batch, Wl1, bl1, Wr1, br1, att1, b1,
           Wl2, bl2, Wr2, br2, att2, b2, Wlin, blin):
    src3 = jnp.reshape(edge_index[0], (NTILES, NCH, B))
    dst3 = jnp.reshape(edge_index[1], (NTILES, NCH, B))
    batchr = jnp.reshape(batch, (NBLK, 1, R))

    bl1r = jnp.reshape(bl1, (1, 16))
    br1r = jnp.reshape(br1, (1, 16))
    b1r = jnp.reshape(b1, (1, 16))
    att1r = jnp.reshape(att1, (1, 16))
    b2r = jnp.reshape(jnp.pad(b2, (0, 8)), (1, 16))
    att2p = jnp.pad(att2, (0, 8))
    att2r = jnp.reshape(att2p, (1, 16))
    Wl2p = jnp.pad(Wl2, ((0, 0), (0, 8)))
    Wr2p = jnp.pad(Wr2, ((0, 0), (0, 8)))
    bl2r = jnp.reshape(jnp.pad(bl2, (0, 8)), (1, 16))
    br2r = jnp.reshape(jnp.pad(br2, (0, 8)), (1, 16))
    Wlinp = jnp.pad(Wlin, ((0, 8), (0, 0)))
    blinr = jnp.reshape(blin, (1, 1))

    xl1, xr1 = _proj(x, Wl1, bl1r, Wr1, br1r)
    outd1 = _sc_agg(xl1, xr1, src3, dst3, att1)
    xl2, xr2 = _norm_proj(outd1, xl1, xr1, att1r, b1r, Wl2p, bl2r, Wr2p, br2r)
    outd2 = _sc_agg(xl2, xr2, src3, dst3, att2p)
    return _norm_pool(outd2, xl2, xr2, att2r, b2r, batchr, Wlinp, blinr)
